# baseline (speedup 1.0000x reference)
import jax
import jax.numpy as jnp
from jax.experimental import pallas as pl
from jax.experimental.pallas import tpu as pltpu

CHUNK = 200
NBUF = 4


def _gcn_body(adj_hbm, x_ref, w_ref, out_ref, buf_ref, support_ref, sems):
    n = adj_hbm.shape[0]
    nchunks = n // CHUNK

    def copy_chunk(i, j):
        return pltpu.make_async_copy(
            adj_hbm.at[pl.ds(i * CHUNK, CHUNK), :],
            buf_ref.at[j],
            sems.at[j],
        )

    for j in range(NBUF):
        copy_chunk(j, j).start()

    support_ref[...] = jnp.dot(
        x_ref[...], w_ref[...], preferred_element_type=jnp.float32
    )

    def step(i, carry):
        j = jax.lax.rem(i, NBUF)
        copy_chunk(i, j).wait()
        out_ref[pl.ds(i * CHUNK, CHUNK), :] = jnp.dot(
            buf_ref[j], support_ref[...], preferred_element_type=jnp.float32
        )

        @pl.when(i + NBUF < nchunks)
        def _():
            copy_chunk(i + NBUF, j).start()

        return carry

    jax.lax.fori_loop(0, nchunks, step, 0)


def kernel(adj, x, weight):
    n, d = x.shape
    return pl.pallas_call(
        _gcn_body,
        in_specs=[
            pl.BlockSpec(memory_space=pl.ANY),
            pl.BlockSpec(memory_space=pltpu.MemorySpace.VMEM),
            pl.BlockSpec(memory_space=pltpu.MemorySpace.VMEM),
        ],
        out_specs=pl.BlockSpec(memory_space=pltpu.MemorySpace.VMEM),
        out_shape=jax.ShapeDtypeStruct((n, d), jnp.float32),
        scratch_shapes=[
            pltpu.VMEM((NBUF, CHUNK, n), jnp.float32),
            pltpu.VMEM((n, d), jnp.float32),
            pltpu.SemaphoreType.DMA((NBUF,)),
        ],
    )(adj, x, weight)


# speedup vs baseline: 1.0241x; 1.0241x over previous
import jax
import jax.numpy as jnp
from jax.experimental import pallas as pl
from jax.experimental.pallas import tpu as pltpu

CHUNK = 200
NBUF = 4


def _gcn_body(adj_hbm, x_ref, w_ref, out_ref, buf_ref, support_ref, sems):
    n = adj_hbm.shape[0]
    nchunks = n // CHUNK

    def copy_chunk(i, j):
        return pltpu.make_async_copy(
            adj_hbm.at[pl.ds(i * CHUNK, CHUNK), :],
            buf_ref.at[j],
            sems.at[j],
        )

    for j in range(NBUF):
        copy_chunk(j, j).start()

    support_ref[...] = jnp.dot(
        x_ref[...], w_ref[...], preferred_element_type=jnp.float32
    )

    for i in range(nchunks):
        j = i % NBUF
        copy_chunk(i, j).wait()
        out_ref[i * CHUNK : (i + 1) * CHUNK, :] = jnp.dot(
            buf_ref[j], support_ref[...], preferred_element_type=jnp.float32
        )
        if i + NBUF < nchunks:
            copy_chunk(i + NBUF, j).start()


def kernel(adj, x, weight):
    n, d = x.shape
    return pl.pallas_call(
        _gcn_body,
        in_specs=[
            pl.BlockSpec(memory_space=pl.ANY),
            pl.BlockSpec(memory_space=pltpu.MemorySpace.VMEM),
            pl.BlockSpec(memory_space=pltpu.MemorySpace.VMEM),
        ],
        out_specs=pl.BlockSpec(memory_space=pltpu.MemorySpace.VMEM),
        out_shape=jax.ShapeDtypeStruct((n, d), jnp.float32),
        scratch_shapes=[
            pltpu.VMEM((NBUF, CHUNK, n), jnp.float32),
            pltpu.VMEM((n, d), jnp.float32),
            pltpu.SemaphoreType.DMA((NBUF,)),
        ],
    )(adj, x, weight)
